# baseline (speedup 1.0000x reference)
import functools

import jax
import jax.numpy as jnp
from jax import lax
from jax.experimental import pallas as pl
from jax.experimental.pallas import tpu as pltpu
from jax.experimental.pallas import tpu_sc as plsc

V_DIM = 1_000_000
EMB = 64
BATCH = 16384
CTX = 20
LANES = 16
ROW_W = 128

NC = 2
NS = 16
NW = NC * NS

B_PER_W = BATCH // NW
T = 32
NCHUNK = B_PER_W // T
IDX_W = 128
IDX_ROWS = B_PER_W * CTX // IDX_W
ROWS_PER_CHUNK = T * CTX
DMA_PER_CHUNK = ROWS_PER_CHUNK // IDX_W


def _cbow_body(x_hbm, table_hbm, out_hbm, idx_v, rows_v, outc_v, sem):
    wid = lax.axis_index("s") * NC + lax.axis_index("c")
    pltpu.sync_copy(x_hbm.at[pl.ds(wid * IDX_ROWS, IDX_ROWS), :], idx_v)
    out_base = wid * B_PER_W

    def chunk_body(t, carry):
        cps = []
        for j in range(DMA_PER_CHUNK):
            cps.append(
                pltpu.async_copy(
                    table_hbm.at[idx_v.at[t * DMA_PER_CHUNK + j]],
                    rows_v.at[pl.ds(j * IDX_W, IDX_W), :],
                    sem,
                )
            )
        for cp in cps:
            cp.wait()

        def b_body(b, bcarry):
            r0 = b * CTX
            for k in range(EMB // LANES):
                sl = pl.ds(k * LANES, LANES)
                a0 = rows_v[r0 + 0, sl] + rows_v[r0 + 1, sl]
                a1 = rows_v[r0 + 2, sl] + rows_v[r0 + 3, sl]
                a2 = rows_v[r0 + 4, sl] + rows_v[r0 + 5, sl]
                a3 = rows_v[r0 + 6, sl] + rows_v[r0 + 7, sl]
                for c in range(8, CTX, 4):
                    a0 = a0 + rows_v[r0 + c + 0, sl]
                    a1 = a1 + rows_v[r0 + c + 1, sl]
                    a2 = a2 + rows_v[r0 + c + 2, sl]
                    a3 = a3 + rows_v[r0 + c + 3, sl]
                outc_v[b, sl] = ((a0 + a1) + (a2 + a3)) * (1.0 / CTX)
            return bcarry

        lax.fori_loop(0, T, b_body, 0)
        pltpu.sync_copy(outc_v, out_hbm.at[pl.ds(out_base + t * T, T), :])
        return carry

    lax.fori_loop(0, NCHUNK, chunk_body, 0)


def kernel(x, table):
    tblp = jnp.pad(table, ((0, 0), (0, ROW_W - EMB)))
    x2 = x.reshape(BATCH * CTX // IDX_W, IDX_W)
    mesh = plsc.VectorSubcoreMesh(core_axis_name="c", subcore_axis_name="s")
    run = functools.partial(
        pl.kernel,
        mesh=mesh,
        out_type=jax.ShapeDtypeStruct((BATCH, EMB), jnp.float32),
        scratch_types=[
            pltpu.VMEM((IDX_ROWS, IDX_W), jnp.int32),
            pltpu.VMEM((ROWS_PER_CHUNK, ROW_W), jnp.float32),
            pltpu.VMEM((T, EMB), jnp.float32),
            pltpu.SemaphoreType.DMA,
        ],
    )(_cbow_body)
    return run(x2, tblp)
